# Initial kernel scaffold; baseline (speedup 1.0000x reference)
import jax
import jax.numpy as jnp
from jax.experimental import pallas as pl


def kernel(latents, embedding):
    raise NotImplementedError("write your pallas kernel here")


# speedup vs baseline: 1.4282x; 1.4282x over previous
import functools

import jax
import jax.numpy as jnp
from jax import lax
from jax.experimental import pallas as pl
from jax.experimental.pallas import tpu as pltpu
from jax.experimental.pallas import tpu_sc as plsc

_BETA = 0.25
_BN = 512


def _tc_body(x_ref, e_ref, inds_ref, loss_ref, *, kk):
    x = x_ref[...]
    e = e_ref[...]
    xn = jnp.sum(x * x, axis=1, keepdims=True)
    en = jnp.sum(e * e, axis=1)
    xe = lax.dot_general(x, e, (((1,), (1,)), ((), ())),
                         preferred_element_type=jnp.float32)
    dist = (xn + en[None, :]) - 2.0 * xe
    m = jnp.min(dist, axis=1, keepdims=True)
    kio = lax.broadcasted_iota(jnp.int32, dist.shape, 1)
    inds = jnp.min(jnp.where(dist == m, kio, kk), axis=1, keepdims=True)
    inds_ref[...] = inds
    loss_ref[...] = (1.0 + _BETA) * m


def _tc_dist_argmin(latents, embedding):
    n, d = latents.shape
    kk = embedding.shape[0]
    grid = n // _BN
    inds2d, loss2d = pl.pallas_call(
        functools.partial(_tc_body, kk=kk),
        grid=(grid,),
        in_specs=[
            pl.BlockSpec((_BN, d), lambda i: (i, 0)),
            pl.BlockSpec((kk, d), lambda i: (0, 0)),
        ],
        out_specs=[
            pl.BlockSpec((_BN, 1), lambda i: (i, 0)),
            pl.BlockSpec((_BN, 1), lambda i: (i, 0)),
        ],
        out_shape=[
            jax.ShapeDtypeStruct((n, 1), jnp.int32),
            jax.ShapeDtypeStruct((n, 1), jnp.float32),
        ],
    )(latents, embedding)
    return inds2d.reshape(n), loss2d.reshape(n)


def _sc_gather(table, idx):
    n = idx.shape[0]
    d = table.shape[1]
    info = plsc.get_sparse_core_info()
    nc, ns = info.num_cores, info.num_subcores
    nw = nc * ns
    b_per_w = n // nw
    mesh = plsc.VectorSubcoreMesh(core_axis_name="c", subcore_axis_name="s")

    @functools.partial(
        pl.kernel,
        out_type=jax.ShapeDtypeStruct((n, d), jnp.float32),
        mesh=mesh,
        scratch_types=[
            pltpu.VMEM((b_per_w,), jnp.int32),
            pltpu.VMEM((b_per_w, d), jnp.float32),
            pltpu.SemaphoreType.DMA,
        ],
        compiler_params=pltpu.CompilerParams(use_tc_tiling_on_sc=False),
    )
    def gather(table_hbm, idx_hbm, out_hbm, idx_v, rows_v, sem):
        wid = lax.axis_index("s") * nc + lax.axis_index("c")
        base = wid * b_per_w
        pltpu.sync_copy(idx_hbm.at[pl.ds(base, b_per_w)], idx_v)
        pltpu.async_copy(table_hbm.at[idx_v], rows_v, sem).wait()
        pltpu.sync_copy(rows_v, out_hbm.at[pl.ds(base, b_per_w)])

    return gather(table, idx)


def kernel(latents, embedding):
    inds, vq_loss = _tc_dist_argmin(latents, embedding)
    quantized = _sc_gather(embedding, inds)
    return quantized, vq_loss


# speedup vs baseline: 1.4422x; 1.0098x over previous
import functools

import jax
import jax.numpy as jnp
from jax import lax
from jax.experimental import pallas as pl
from jax.experimental.pallas import tpu as pltpu
from jax.experimental.pallas import tpu_sc as plsc

_BETA = 0.25
_BN = 512


def _tc_body(x_ref, e_ref, inds_ref, loss_ref, *, kk):
    x = x_ref[...]
    e = e_ref[...]
    xn = jnp.sum(x * x, axis=1, keepdims=True)
    en = jnp.sum(e * e, axis=1)
    xe = lax.dot_general(x, e, (((1,), (1,)), ((), ())),
                         preferred_element_type=jnp.float32)
    dist = (xn + en[None, :]) - 2.0 * xe
    m = jnp.min(dist, axis=1, keepdims=True)
    kio = lax.broadcasted_iota(jnp.int32, dist.shape, 1)
    inds = jnp.min(jnp.where(dist == m, kio, kk), axis=1, keepdims=True)
    inds_ref[...] = inds
    loss_ref[...] = (1.0 + _BETA) * m


def _tc_dist_argmin(latents, embedding):
    n, d = latents.shape
    kk = embedding.shape[0]
    grid = n // _BN
    inds2d, loss2d = pl.pallas_call(
        functools.partial(_tc_body, kk=kk),
        grid=(grid,),
        in_specs=[
            pl.BlockSpec((_BN, d), lambda i: (i, 0)),
            pl.BlockSpec((kk, d), lambda i: (0, 0)),
        ],
        out_specs=[
            pl.BlockSpec((_BN, 1), lambda i: (i, 0)),
            pl.BlockSpec((_BN, 1), lambda i: (i, 0)),
        ],
        out_shape=[
            jax.ShapeDtypeStruct((n, 1), jnp.int32),
            jax.ShapeDtypeStruct((n, 1), jnp.float32),
        ],
    )(latents, embedding)
    return inds2d.reshape(n), loss2d.reshape(n)


def _sc_gather(table, idx):
    n = idx.shape[0]
    dp = table.shape[1]
    info = plsc.get_sparse_core_info()
    nc, ns = info.num_cores, info.num_subcores
    nw = nc * ns
    b_per_w = n // nw
    chunk = b_per_w // 2
    mesh = plsc.VectorSubcoreMesh(core_axis_name="c", subcore_axis_name="s")

    @functools.partial(
        pl.kernel,
        out_type=jax.ShapeDtypeStruct((n, dp), jnp.float32),
        mesh=mesh,
        scratch_types=[
            pltpu.VMEM((b_per_w,), jnp.int32),
            pltpu.VMEM((chunk, dp), jnp.float32),
            pltpu.SemaphoreType.DMA,
        ],
    )
    def gather(table_hbm, idx_hbm, out_hbm, idx_v, rows_v, sem):
        wid = lax.axis_index("s") * nc + lax.axis_index("c")
        base = wid * b_per_w
        pltpu.sync_copy(idx_hbm.at[pl.ds(base, b_per_w)], idx_v)
        for c in range(2):
            pltpu.async_copy(table_hbm.at[idx_v.at[pl.ds(c * chunk, chunk)]],
                             rows_v, sem).wait()
            pltpu.sync_copy(rows_v, out_hbm.at[pl.ds(base + c * chunk, chunk)])

    return gather(table, idx)


def kernel(latents, embedding):
    inds, vq_loss = _tc_dist_argmin(latents, embedding)
    d = embedding.shape[1]
    table = jnp.pad(embedding, ((0, 0), (0, 128 - d)))
    quantized = _sc_gather(table, inds)[:, :d]
    return quantized, vq_loss
